# baseline (speedup 1.0000x reference)
import functools

import jax
import jax.numpy as jnp
from jax import lax
from jax.experimental import pallas as pl
from jax.experimental.pallas import tpu as pltpu
from jax.experimental.pallas import tpu_sc as plsc

C_CHUNK = 16


def _prelude_body(x_ref, y_ref, idx_ref, w_ref, *, s):
    b = pl.program_id(0)
    x = x_ref[0]
    y = y_ref[0]
    sm1 = jnp.float32(s - 1)

    def norm(v):
        v = v - jnp.min(v)
        return v * (sm1 / jnp.max(v))

    xs = norm(x)
    ys = norm(y)
    x1f = jnp.floor(xs)
    y1f = jnp.floor(ys)
    x1 = x1f.astype(jnp.int32)
    y1 = y1f.astype(jnp.int32)
    x2 = jnp.minimum(jnp.ceil(xs).astype(jnp.int32), s - 1)
    y2 = jnp.minimum(jnp.ceil(ys).astype(jnp.int32), s - 1)
    x2f = x2.astype(jnp.float32)
    y2f = y2.astype(jnp.float32)
    wx1 = x2f - xs
    wx2 = xs - x1f
    wy1 = y2f - ys
    wy2 = ys - y1f
    base = b * (s * s)
    idx = [x1 * s + y1 + base, x1 * s + y2 + base,
           x2 * s + y1 + base, x2 * s + y2 + base]
    w = [wx1 * wy1, wx1 * wy2, wx2 * wy1, wx2 * wy2]
    for j in range(4):
        idx_ref[j, 0] = idx[j]
        w_ref[j, 0] = w[j]


def _prelude(x2d, y2d, *, nb, n, s):
    kern = functools.partial(_prelude_body, s=s)
    return pl.pallas_call(
        kern,
        grid=(nb,),
        in_specs=[pl.BlockSpec((1, 1, n), lambda b: (b, 0, 0)),
                  pl.BlockSpec((1, 1, n), lambda b: (b, 0, 0))],
        out_specs=[pl.BlockSpec((4, 1, 1, n), lambda b: (0, b, 0, 0)),
                   pl.BlockSpec((4, 1, 1, n), lambda b: (0, b, 0, 0))],
        out_shape=[jax.ShapeDtypeStruct((4, nb, 1, n), jnp.int32),
                   jax.ShapeDtypeStruct((4, nb, 1, n), jnp.float32)],
    )(x2d, y2d)


def _sc_gather_combine(table, idx, wrep, *, nb, n, ch):
    info = plsc.get_sparse_core_info()
    nc, ns = info.num_cores, info.num_subcores
    nw = nc * ns
    total = nb * n
    base_chunks = total // C_CHUNK
    ntile_max = -(-base_chunks // nw) * C_CHUNK

    nbuf = 2
    chh = ch // 2
    ngr = chh // 16
    WREP = 64

    @functools.partial(
        pl.kernel,
        mesh=plsc.VectorSubcoreMesh(core_axis_name="c", subcore_axis_name="s"),
        out_type=jax.ShapeDtypeStruct((total * ch,), jnp.float32),
        scratch_types=[
            pltpu.VMEM((ntile_max,), jnp.int32),
            pltpu.VMEM((ntile_max,), jnp.int32),
            pltpu.VMEM((ntile_max,), jnp.int32),
            pltpu.VMEM((ntile_max,), jnp.int32),
            pltpu.VMEM((C_CHUNK, chh), jnp.int32),
            pltpu.VMEM((C_CHUNK, chh), jnp.int32),
            pltpu.VMEM((C_CHUNK, chh), jnp.int32),
            pltpu.VMEM((C_CHUNK, chh), jnp.int32),
            pltpu.VMEM((C_CHUNK, chh), jnp.int32),
            pltpu.VMEM((C_CHUNK, chh), jnp.int32),
            pltpu.VMEM((C_CHUNK, chh), jnp.int32),
            pltpu.VMEM((C_CHUNK, chh), jnp.int32),
            pltpu.VMEM((C_CHUNK * WREP,), jnp.float32),
            pltpu.VMEM((C_CHUNK * WREP,), jnp.float32),
            pltpu.VMEM((C_CHUNK * ch,), jnp.float32),
            pltpu.VMEM((C_CHUNK * ch,), jnp.float32),
            pltpu.SemaphoreType.DMA,
            pltpu.SemaphoreType.DMA,
            pltpu.SemaphoreType.DMA,
            pltpu.SemaphoreType.DMA,
        ],
    )
    def k(table_hbm, idx_hbm, w_hbm, out_hbm,
          i0_v, i1_v, i2_v, i3_v,
          ra0, ra1, ra2, ra3, rb0, rb1, rb2, rb3, wa, wb, oa, ob,
          sem0, sem1, osem0, osem1):
        wid = lax.axis_index("s") * nc + lax.axis_index("c")
        start = C_CHUNK * ((wid * base_chunks) // nw)
        nck = ((wid + 1) * base_chunks) // nw - (wid * base_chunks) // nw
        ivs = (i0_v, i1_v, i2_v, i3_v)
        for j in range(4):
            pltpu.sync_copy(idx_hbm.at[pl.ds(j * total + start, ntile_max)],
                            ivs[j])
        rows = ((ra0, ra1, ra2, ra3), (rb0, rb1, rb2, rb3))
        wbufs = (wa, wb)
        sems = (sem0, sem1)
        outs = (oa, ob)
        osems = (osem0, osem1)

        def issue(c, b):
            for j in range(4):
                pltpu.async_copy(
                    table_hbm.at[ivs[j].at[pl.ds(c * C_CHUNK, C_CHUNK)]],
                    rows[b][j], sems[b])
            pltpu.async_copy(
                w_hbm.at[pl.ds((start + c * C_CHUNK) * WREP, C_CHUNK * WREP)],
                wbufs[b], sems[b])

        def drain(b):
            for j in range(4):
                pltpu.make_async_copy(table_hbm.at[pl.ds(0, C_CHUNK)],
                                      rows[b][j], sems[b]).wait()
            pltpu.make_async_copy(w_hbm.at[pl.ds(0, C_CHUNK * WREP)],
                                  wbufs[b], sems[b]).wait()

        def drain_store(b):
            pltpu.make_async_copy(outs[b],
                                  out_hbm.at[pl.ds(0, C_CHUNK * ch)],
                                  osems[b]).wait()

        def combine(c, b):
            r0, r1, r2, r3 = rows[b]
            wbuf = wbufs[b]
            o = outs[b]

            f32 = jnp.float32
            hi_mask = jnp.int32(-65536)

            def unpack(v):
                lo = lax.bitcast_convert_type(v << 16, f32)
                hi = lax.bitcast_convert_type(v & hi_mask, f32)
                return lo, hi

            for i in range(C_CHUNK):
                wv = [wbuf[pl.ds(i * WREP + j * 16, 16)] for j in range(4)]
                for g in range(ngr):
                    sl = pl.ds(g * 16, 16)
                    l0, h0 = unpack(r0[i, sl])
                    l1, h1 = unpack(r1[i, sl])
                    l2, h2 = unpack(r2[i, sl])
                    l3, h3 = unpack(r3[i, sl])
                    acc_lo = (l0 * wv[0] + l2 * wv[2]
                              + l1 * wv[1] + l3 * wv[3])
                    acc_hi = (h0 * wv[0] + h2 * wv[2]
                              + h1 * wv[1] + h3 * wv[3])
                    o[pl.ds(i * ch + g * 16, 16)] = acc_lo
                    o[pl.ds(i * ch + chh + g * 16, 16)] = acc_hi

        issue(0, 0)

        @pl.when(nck > 1)
        def _():
            issue(1, 1)

        def do_group(g, carry):
            for b in range(nbuf):
                c = g * nbuf + b

                @pl.when(c < nck)
                def _(c=c, b=b):
                    drain(b)

                    @pl.when(c >= nbuf)
                    def _():
                        drain_store(b)

                    combine(c, b)

                    @pl.when(c + nbuf < nck)
                    def _():
                        issue(c + nbuf, b)
                    pltpu.async_copy(
                        outs[b],
                        out_hbm.at[pl.ds((start + c * C_CHUNK) * ch,
                                         C_CHUNK * ch)],
                        osems[b])
            return carry

        lax.fori_loop(0, (nck + nbuf - 1) // nbuf, do_group, 0)
        drain_store(0)

        @pl.when(nck > 1)
        def _():
            drain_store(1)

    return k(table, idx, wrep)


def kernel(img_features, points, R, T):
    nb, ch, s, s2 = img_features.shape
    n = points.shape[1]
    assert s == s2 and ch % 32 == 0 and n % C_CHUNK == 0

    points2d = jnp.einsum('bnd,de->bne', points, R) + T
    x2d = points2d[:, :, 0].reshape(nb, 1, n)
    y2d = points2d[:, :, 1].reshape(nb, 1, n)

    idx, w = _prelude(x2d, y2d, nb=nb, n=n, s=s)
    idx1d = idx.reshape(4 * nb * n)
    wrep = jnp.repeat(
        w.reshape(4, nb * n).T, 16, axis=1,
    ).reshape(nb * n * 64)
    t16 = img_features.transpose(0, 2, 3, 1).reshape(nb * s * s2, ch)
    t16 = lax.bitcast_convert_type(
        t16.astype(jnp.bfloat16), jnp.uint16).astype(jnp.uint32)
    table = lax.bitcast_convert_type(
        t16[:, :ch // 2] | (t16[:, ch // 2:] << 16), jnp.int32)
    out = _sc_gather_combine(table, idx1d, wrep, nb=nb, n=n, ch=ch)
    return out.reshape(nb, n, ch)


# speedup vs baseline: 1.2068x; 1.2068x over previous
import functools

import jax
import jax.numpy as jnp
from jax import lax
from jax.experimental import pallas as pl
from jax.experimental.pallas import tpu as pltpu
from jax.experimental.pallas import tpu_sc as plsc

C_CHUNK = 16


def _prelude_body(x_ref, y_ref, idx_ref, w_ref, *, s):
    b = pl.program_id(0)
    x = x_ref[0]
    y = y_ref[0]
    sm1 = jnp.float32(s - 1)

    def norm(v):
        v = v - jnp.min(v)
        return v * (sm1 / jnp.max(v))

    xs = norm(x)
    ys = norm(y)
    x1f = jnp.floor(xs)
    y1f = jnp.floor(ys)
    x1 = x1f.astype(jnp.int32)
    y1 = y1f.astype(jnp.int32)
    x2 = jnp.minimum(jnp.ceil(xs).astype(jnp.int32), s - 1)
    y2 = jnp.minimum(jnp.ceil(ys).astype(jnp.int32), s - 1)
    x2f = x2.astype(jnp.float32)
    y2f = y2.astype(jnp.float32)
    wx1 = x2f - xs
    wx2 = xs - x1f
    wy1 = y2f - ys
    wy2 = ys - y1f
    base = b * (s * s)
    idx = [x1 * s + y1 + base, x1 * s + y2 + base,
           x2 * s + y1 + base, x2 * s + y2 + base]
    w = [wx1 * wy1, wx1 * wy2, wx2 * wy1, wx2 * wy2]
    for j in range(4):
        idx_ref[j, 0] = idx[j]
        w_ref[j, 0] = w[j]


def _prelude(x2d, y2d, *, nb, n, s):
    kern = functools.partial(_prelude_body, s=s)
    return pl.pallas_call(
        kern,
        grid=(nb,),
        in_specs=[pl.BlockSpec((1, 1, n), lambda b: (b, 0, 0)),
                  pl.BlockSpec((1, 1, n), lambda b: (b, 0, 0))],
        out_specs=[pl.BlockSpec((4, 1, 1, n), lambda b: (0, b, 0, 0)),
                   pl.BlockSpec((4, 1, 1, n), lambda b: (0, b, 0, 0))],
        out_shape=[jax.ShapeDtypeStruct((4, nb, 1, n), jnp.int32),
                   jax.ShapeDtypeStruct((4, nb, 1, n), jnp.float32)],
    )(x2d, y2d)


def _sc_gather_combine(table, idx, wrep, *, nb, n, ch):
    info = plsc.get_sparse_core_info()
    nc, ns = info.num_cores, info.num_subcores
    nw = nc * ns
    total = nb * n
    base_chunks = total // C_CHUNK
    ntile_max = -(-base_chunks // nw) * C_CHUNK

    nbuf = 2
    chh = ch // 2
    ngr = chh // 16
    WREP = 64

    @functools.partial(
        pl.kernel,
        mesh=plsc.VectorSubcoreMesh(core_axis_name="c", subcore_axis_name="s"),
        out_type=jax.ShapeDtypeStruct((total * ch,), jnp.float32),
        scratch_types=[
            pltpu.VMEM((ntile_max,), jnp.int32),
            pltpu.VMEM((ntile_max,), jnp.int32),
            pltpu.VMEM((ntile_max,), jnp.int32),
            pltpu.VMEM((ntile_max,), jnp.int32),
            pltpu.VMEM((C_CHUNK, chh), jnp.int32),
            pltpu.VMEM((C_CHUNK, chh), jnp.int32),
            pltpu.VMEM((C_CHUNK, chh), jnp.int32),
            pltpu.VMEM((C_CHUNK, chh), jnp.int32),
            pltpu.VMEM((C_CHUNK, chh), jnp.int32),
            pltpu.VMEM((C_CHUNK, chh), jnp.int32),
            pltpu.VMEM((C_CHUNK, chh), jnp.int32),
            pltpu.VMEM((C_CHUNK, chh), jnp.int32),
            pltpu.VMEM((C_CHUNK * WREP,), jnp.float32),
            pltpu.VMEM((C_CHUNK * WREP,), jnp.float32),
            pltpu.VMEM((C_CHUNK * ch,), jnp.float32),
            pltpu.VMEM((C_CHUNK * ch,), jnp.float32),
            pltpu.SemaphoreType.DMA,
            pltpu.SemaphoreType.DMA,
            pltpu.SemaphoreType.DMA,
            pltpu.SemaphoreType.DMA,
        ],
    )
    def k(table_hbm, idx_hbm, w_hbm, out_hbm,
          i0_v, i1_v, i2_v, i3_v,
          ra0, ra1, ra2, ra3, rb0, rb1, rb2, rb3, wa, wb, oa, ob,
          sem0, sem1, osem0, osem1):
        wid = lax.axis_index("s") * nc + lax.axis_index("c")
        start = C_CHUNK * ((wid * base_chunks) // nw)
        nck = ((wid + 1) * base_chunks) // nw - (wid * base_chunks) // nw
        ivs = (i0_v, i1_v, i2_v, i3_v)
        for j in range(4):
            pltpu.sync_copy(idx_hbm.at[pl.ds(j * total + start, ntile_max)],
                            ivs[j])
        rows = ((ra0, ra1, ra2, ra3), (rb0, rb1, rb2, rb3))
        wbufs = (wa, wb)
        sems = (sem0, sem1)
        outs = (oa, ob)
        osems = (osem0, osem1)

        def issue(c, b):
            for j in range(4):
                pltpu.async_copy(
                    table_hbm.at[ivs[j].at[pl.ds(c * C_CHUNK, C_CHUNK)]],
                    rows[b][j], sems[b])
            pltpu.async_copy(
                w_hbm.at[pl.ds((start + c * C_CHUNK) * WREP, C_CHUNK * WREP)],
                wbufs[b], sems[b])

        def drain(b):
            for j in range(4):
                pltpu.make_async_copy(table_hbm.at[pl.ds(0, C_CHUNK)],
                                      rows[b][j], sems[b]).wait()
            pltpu.make_async_copy(w_hbm.at[pl.ds(0, C_CHUNK * WREP)],
                                  wbufs[b], sems[b]).wait()

        def drain_store(b):
            pltpu.make_async_copy(outs[b],
                                  out_hbm.at[pl.ds(0, C_CHUNK * ch)],
                                  osems[b]).wait()

        def combine(c, b):
            r0, r1, r2, r3 = rows[b]
            wbuf = wbufs[b]
            o = outs[b]

            f32 = jnp.float32
            hi_mask = jnp.int32(-65536)

            def unpack(v):
                lo = lax.bitcast_convert_type(v << 16, f32)
                hi = lax.bitcast_convert_type(v & hi_mask, f32)
                return lo, hi

            def body(i, cy):
                wv = [wbuf[pl.ds(i * WREP + j * 16, 16)] for j in range(4)]
                for g in range(ngr):
                    sl = pl.ds(g * 16, 16)
                    l0, h0 = unpack(r0[i, sl])
                    l1, h1 = unpack(r1[i, sl])
                    l2, h2 = unpack(r2[i, sl])
                    l3, h3 = unpack(r3[i, sl])
                    acc_lo = (l0 * wv[0] + l2 * wv[2]
                              + l1 * wv[1] + l3 * wv[3])
                    acc_hi = (h0 * wv[0] + h2 * wv[2]
                              + h1 * wv[1] + h3 * wv[3])
                    o[pl.ds(i * ch + g * 16, 16)] = acc_lo
                    o[pl.ds(i * ch + chh + g * 16, 16)] = acc_hi
                return cy

            lax.fori_loop(0, C_CHUNK, body, 0)

        issue(0, 0)

        @pl.when(nck > 1)
        def _():
            issue(1, 1)

        def do_group(g, carry):
            for b in range(nbuf):
                c = g * nbuf + b

                @pl.when(c < nck)
                def _(c=c, b=b):
                    drain(b)

                    @pl.when(c >= nbuf)
                    def _():
                        drain_store(b)

                    combine(c, b)

                    @pl.when(c + nbuf < nck)
                    def _():
                        issue(c + nbuf, b)
                    pltpu.async_copy(
                        outs[b],
                        out_hbm.at[pl.ds((start + c * C_CHUNK) * ch,
                                         C_CHUNK * ch)],
                        osems[b])
            return carry

        lax.fori_loop(0, (nck + nbuf - 1) // nbuf, do_group, 0)
        drain_store(0)

        @pl.when(nck > 1)
        def _():
            drain_store(1)

    return k(table, idx, wrep)


def kernel(img_features, points, R, T):
    nb, ch, s, s2 = img_features.shape
    n = points.shape[1]
    assert s == s2 and ch % 32 == 0 and n % C_CHUNK == 0

    points2d = jnp.einsum('bnd,de->bne', points, R) + T
    x2d = points2d[:, :, 0].reshape(nb, 1, n)
    y2d = points2d[:, :, 1].reshape(nb, 1, n)

    idx, w = _prelude(x2d, y2d, nb=nb, n=n, s=s)
    idx1d = idx.reshape(4 * nb * n)
    wrep = jnp.repeat(
        w.reshape(4, nb * n).T, 16, axis=1,
    ).reshape(nb * n * 64)
    t16 = img_features.transpose(0, 2, 3, 1).reshape(nb * s * s2, ch)
    t16 = lax.bitcast_convert_type(
        t16.astype(jnp.bfloat16), jnp.uint16).astype(jnp.uint32)
    table = lax.bitcast_convert_type(
        t16[:, :ch // 2] | (t16[:, ch // 2:] << 16), jnp.int32)
    out = _sc_gather_combine(table, idx1d, wrep, nb=nb, n=n, ch=ch)
    return out.reshape(nb, n, ch)


# speedup vs baseline: 2.7307x; 2.2627x over previous
import functools

import jax
import jax.numpy as jnp
from jax import lax
from jax.experimental import pallas as pl
from jax.experimental.pallas import tpu as pltpu
from jax.experimental.pallas import tpu_sc as plsc

C_CHUNK = 16


def _prelude_body(x_ref, y_ref, idx_ref, w_ref, *, s):
    b = pl.program_id(0)
    x = x_ref[0]
    y = y_ref[0]
    sm1 = jnp.float32(s - 1)

    def norm(v):
        v = v - jnp.min(v)
        return v * (sm1 / jnp.max(v))

    xs = norm(x)
    ys = norm(y)
    x1f = jnp.floor(xs)
    y1f = jnp.floor(ys)
    x1 = x1f.astype(jnp.int32)
    y1 = y1f.astype(jnp.int32)
    x2 = jnp.minimum(jnp.ceil(xs).astype(jnp.int32), s - 1)
    y2 = jnp.minimum(jnp.ceil(ys).astype(jnp.int32), s - 1)
    x2f = x2.astype(jnp.float32)
    y2f = y2.astype(jnp.float32)
    wx1 = x2f - xs
    wx2 = xs - x1f
    wy1 = y2f - ys
    wy2 = ys - y1f
    base = b * (s * s)
    idx = [x1 * s + y1 + base, x1 * s + y2 + base,
           x2 * s + y1 + base, x2 * s + y2 + base]
    w = [wx1 * wy1, wx1 * wy2, wx2 * wy1, wx2 * wy2]
    for j in range(4):
        idx_ref[j, 0] = idx[j]
        w_ref[j, 0] = w[j]


def _prelude(x2d, y2d, *, nb, n, s):
    kern = functools.partial(_prelude_body, s=s)
    return pl.pallas_call(
        kern,
        grid=(nb,),
        in_specs=[pl.BlockSpec((1, 1, n), lambda b: (b, 0, 0)),
                  pl.BlockSpec((1, 1, n), lambda b: (b, 0, 0))],
        out_specs=[pl.BlockSpec((4, 1, 1, n), lambda b: (0, b, 0, 0)),
                   pl.BlockSpec((4, 1, 1, n), lambda b: (0, b, 0, 0))],
        out_shape=[jax.ShapeDtypeStruct((4, nb, 1, n), jnp.int32),
                   jax.ShapeDtypeStruct((4, nb, 1, n), jnp.float32)],
    )(x2d, y2d)


def _sc_gather_combine(table, idx, w, *, nb, n, ch):
    info = plsc.get_sparse_core_info()
    nc, ns = info.num_cores, info.num_subcores
    nw = nc * ns
    total = nb * n
    base_chunks = total // C_CHUNK
    ntile_max = -(-base_chunks // nw) * C_CHUNK
    ngr = ch // 16

    nbuf = 2

    @functools.partial(
        pl.kernel,
        mesh=plsc.VectorSubcoreMesh(core_axis_name="c", subcore_axis_name="s"),
        out_type=jax.ShapeDtypeStruct((total, ch), jnp.float32),
        scratch_types=[
            pltpu.VMEM((ntile_max,), jnp.int32),
            pltpu.VMEM((ntile_max,), jnp.int32),
            pltpu.VMEM((ntile_max,), jnp.int32),
            pltpu.VMEM((ntile_max,), jnp.int32),
            pltpu.VMEM((ntile_max + C_CHUNK,), jnp.float32),
            pltpu.VMEM((ntile_max + C_CHUNK,), jnp.float32),
            pltpu.VMEM((ntile_max + C_CHUNK,), jnp.float32),
            pltpu.VMEM((ntile_max + C_CHUNK,), jnp.float32),
            pltpu.VMEM((C_CHUNK, ch), jnp.float32),
            pltpu.VMEM((C_CHUNK, ch), jnp.float32),
            pltpu.VMEM((C_CHUNK, ch), jnp.float32),
            pltpu.VMEM((C_CHUNK, ch), jnp.float32),
            pltpu.VMEM((C_CHUNK, ch), jnp.float32),
            pltpu.VMEM((C_CHUNK, ch), jnp.float32),
            pltpu.VMEM((C_CHUNK, ch), jnp.float32),
            pltpu.VMEM((C_CHUNK, ch), jnp.float32),
            pltpu.SemaphoreType.DMA,
            pltpu.SemaphoreType.DMA,
        ],
    )
    def k(table_hbm, idx_hbm, w_hbm, out_hbm,
          i0_v, i1_v, i2_v, i3_v, w0_v, w1_v, w2_v, w3_v,
          ra0, ra1, ra2, ra3, rb0, rb1, rb2, rb3, sem0, sem1):
        wid = lax.axis_index("s") * nc + lax.axis_index("c")
        start = C_CHUNK * ((wid * base_chunks) // nw)
        nck = ((wid + 1) * base_chunks) // nw - (wid * base_chunks) // nw
        ivs = (i0_v, i1_v, i2_v, i3_v)
        wvs = (w0_v, w1_v, w2_v, w3_v)
        for j in range(4):
            pltpu.sync_copy(idx_hbm.at[pl.ds(j * total + start, ntile_max)],
                            ivs[j])
            pltpu.sync_copy(
                w_hbm.at[pl.ds(j * total + start, ntile_max + C_CHUNK)],
                wvs[j])
        rows = ((ra0, ra1, ra2, ra3), (rb0, rb1, rb2, rb3))
        sems = (sem0, sem1)

        def issue(c, b):
            for j in range(4):
                pltpu.async_copy(
                    table_hbm.at[ivs[j].at[pl.ds(c * C_CHUNK, C_CHUNK)]],
                    rows[b][j], sems[b])

        def drain(b):
            for j in range(4):
                pltpu.make_async_copy(table_hbm.at[pl.ds(0, C_CHUNK)],
                                      rows[b][j], sems[b]).wait()

        def combine_store(c, b):
            r0, r1, r2, r3 = rows[b]

            def body(i, cy):
                pbase = c * C_CHUNK + i
                wv = [jnp.full((16,), wvs[j][pl.ds(pbase, 16)][0])
                      for j in range(4)]
                for g in range(ngr):
                    sl = pl.ds(g * 16, 16)
                    acc = (r0[i, sl] * wv[0] + r2[i, sl] * wv[2]
                           + r1[i, sl] * wv[1] + r3[i, sl] * wv[3])
                    r0[i, sl] = acc
                return cy

            lax.fori_loop(0, C_CHUNK, body, 0)
            pltpu.sync_copy(r0,
                            out_hbm.at[pl.ds(start + c * C_CHUNK, C_CHUNK)])

        issue(0, 0)

        @pl.when(nck > 1)
        def _():
            issue(1, 1)

        def do_group(g, carry):
            for b in range(nbuf):
                c = g * nbuf + b

                @pl.when(c < nck)
                def _(c=c, b=b):
                    drain(b)
                    combine_store(c, b)

                    @pl.when(c + nbuf < nck)
                    def _():
                        issue(c + nbuf, b)
            return carry

        lax.fori_loop(0, (nck + nbuf - 1) // nbuf, do_group, 0)

    return k(table, idx, w)


def kernel(img_features, points, R, T):
    nb, ch, s, s2 = img_features.shape
    n = points.shape[1]
    assert s == s2 and ch % 16 == 0 and n % C_CHUNK == 0

    points2d = jnp.einsum('bnd,de->bne', points, R) + T
    x2d = points2d[:, :, 0].reshape(nb, 1, n)
    y2d = points2d[:, :, 1].reshape(nb, 1, n)

    idx, w = _prelude(x2d, y2d, nb=nb, n=n, s=s)
    idx1d = idx.reshape(4 * nb * n)
    w1d = jnp.concatenate(
        [w.reshape(4 * nb * n), jnp.zeros((C_CHUNK,), jnp.float32)])
    table = img_features.transpose(0, 2, 3, 1).reshape(nb * s * s2, ch)
    out = _sc_gather_combine(table, idx1d, w1d, nb=nb, n=n, ch=ch)
    return out.reshape(nb, n, ch)
